# baseline (speedup 1.0000x reference)
import functools

import jax
import jax.numpy as jnp
from jax import lax
from jax.experimental import pallas as pl
from jax.experimental.pallas import tpu as pltpu
from jax.experimental.pallas import tpu_sc as plsc

_IDX_ROW = 128


def _sc_gather(uid2d, mid2d, user_table, movie_table):
    info = plsc.get_sparse_core_info()
    NC, NS = info.num_cores, info.num_subcores
    NW = NC * NS
    n_rows, L = uid2d.shape
    B = n_rows * L
    D = user_table.shape[1]
    rows_per_w = n_rows // NW
    b_per_w = B // NW
    mesh = plsc.VectorSubcoreMesh(core_axis_name="c", subcore_axis_name="s")

    @functools.partial(
        pl.kernel,
        mesh=mesh,
        out_type=[
            jax.ShapeDtypeStruct((B, D), jnp.float32),
            jax.ShapeDtypeStruct((B, D), jnp.float32),
        ],
        scratch_types=[
            pltpu.VMEM((rows_per_w, L), jnp.int32),
            pltpu.VMEM((rows_per_w, L), jnp.int32),
            pltpu.VMEM((b_per_w, D), jnp.float32),
            pltpu.VMEM((b_per_w, D), jnp.float32),
            pltpu.SemaphoreType.DMA,
        ],
        compiler_params=pltpu.CompilerParams(use_tc_tiling_on_sc=False),
    )
    def gather_kernel(uid_hbm, mid_hbm, utab_hbm, mtab_hbm,
                      uout_hbm, mout_hbm,
                      uidx_v, midx_v, urows_v, mrows_v, sem):
        wid = lax.axis_index("s") * NC + lax.axis_index("c")
        rbase = wid * rows_per_w
        base = wid * b_per_w
        pltpu.sync_copy(uid_hbm.at[pl.ds(rbase, rows_per_w)], uidx_v)
        pltpu.sync_copy(mid_hbm.at[pl.ds(rbase, rows_per_w)], midx_v)
        copies = []
        for j in range(rows_per_w):
            copies.append(pltpu.async_copy(
                utab_hbm.at[uidx_v.at[j]], urows_v.at[pl.ds(j * L, L)], sem))
            copies.append(pltpu.async_copy(
                mtab_hbm.at[midx_v.at[j]], mrows_v.at[pl.ds(j * L, L)], sem))
        for c in copies:
            c.wait()
        pltpu.sync_copy(urows_v, uout_hbm.at[pl.ds(base, b_per_w)])
        pltpu.sync_copy(mrows_v, mout_hbm.at[pl.ds(base, b_per_w)])

    return gather_kernel(uid2d, mid2d, user_table, movie_table)


def _mlp_body(u_ref, m_ref, w1u_ref, w1m_ref, b1_ref, w2_ref, b2_ref,
              w3_ref, b3_ref, o_ref):
    h = jnp.dot(u_ref[...], w1u_ref[...], preferred_element_type=jnp.float32)
    h = h + jnp.dot(m_ref[...], w1m_ref[...], preferred_element_type=jnp.float32)
    h = jnp.maximum(h + b1_ref[...], 0.0)
    h = jnp.maximum(
        jnp.dot(h, w2_ref[...], preferred_element_type=jnp.float32)
        + b2_ref[...], 0.0)
    o_ref[...] = (jnp.dot(h, w3_ref[...], preferred_element_type=jnp.float32)
                  + b3_ref[...])


def _tc_mlp(uvec, mvec, W1u, W1m, b1, W2, b2, W3, b3):
    B, D = uvec.shape
    H = W2.shape[0]
    H2 = W2.shape[1]
    BB = 2048
    grid = (B // BB,)
    full = lambda shape: pl.BlockSpec(shape, lambda i: (0, 0))
    return pl.pallas_call(
        _mlp_body,
        grid=grid,
        in_specs=[
            pl.BlockSpec((BB, D), lambda i: (i, 0)),
            pl.BlockSpec((BB, D), lambda i: (i, 0)),
            full((D, H)),
            full((D, H)),
            full((1, H)),
            full((H, H2)),
            full((1, H2)),
            full((H2, 1)),
            full((1, 1)),
        ],
        out_specs=pl.BlockSpec((BB, 1), lambda i: (i, 0)),
        out_shape=jax.ShapeDtypeStruct((B, 1), jnp.float32),
    )(uvec, mvec, W1u, W1m, b1, W2, b2, W3, b3)


def kernel(user_ids, movie_ids, user_table, movie_table, W1, b1, W2, b2, W3, b3):
    D = user_table.shape[1]
    uid2d = user_ids.astype(jnp.int32).reshape(-1, _IDX_ROW)
    mid2d = movie_ids.astype(jnp.int32).reshape(-1, _IDX_ROW)
    uvec, mvec = _sc_gather(uid2d, mid2d, user_table, movie_table)
    out = _tc_mlp(uvec, mvec, W1[:D], W1[D:], b1.reshape(1, -1),
                  W2, b2.reshape(1, -1), W3, b3.reshape(1, 1))
    return out


# speedup vs baseline: 1.1176x; 1.1176x over previous
import functools

import jax
import jax.numpy as jnp
from jax import lax
from jax.experimental import pallas as pl
from jax.experimental.pallas import tpu as pltpu
from jax.experimental.pallas import tpu_sc as plsc

_LANES = 128


def _sc_gather(uid2d, mid2d, utab_pad, mtab_pad):
    info = plsc.get_sparse_core_info()
    NC, NS = info.num_cores, info.num_subcores
    NW = NC * NS
    n_rows, L = uid2d.shape
    B = n_rows * L
    rows_per_w = n_rows // NW
    b_per_w = B // NW
    half = b_per_w // 2
    mesh = plsc.VectorSubcoreMesh(core_axis_name="c", subcore_axis_name="s")

    @functools.partial(
        pl.kernel,
        mesh=mesh,
        out_type=[
            jax.ShapeDtypeStruct((B, _LANES), jnp.float32),
            jax.ShapeDtypeStruct((B, _LANES), jnp.float32),
        ],
        scratch_types=[
            pltpu.VMEM((rows_per_w, L), jnp.int32),
            pltpu.VMEM((rows_per_w, L), jnp.int32),
            pltpu.VMEM((half, _LANES), jnp.float32),
            pltpu.VMEM((half, _LANES), jnp.float32),
            pltpu.SemaphoreType.DMA,
        ],
        compiler_params=pltpu.CompilerParams(use_tc_tiling_on_sc=True),
    )
    def gather_kernel(uid_hbm, mid_hbm, utab_hbm, mtab_hbm,
                      uout_hbm, mout_hbm,
                      uidx_v, midx_v, urows_v, mrows_v, sem):
        wid = lax.axis_index("s") * NC + lax.axis_index("c")
        rbase = wid * rows_per_w
        base = wid * b_per_w
        pltpu.sync_copy(uid_hbm.at[pl.ds(rbase, rows_per_w)], uidx_v)
        pltpu.sync_copy(mid_hbm.at[pl.ds(rbase, rows_per_w)], midx_v)
        chunks_per_half = half // L
        for h in range(2):
            copies = []
            for c in range(chunks_per_half):
                row = h * chunks_per_half + c
                copies.append(pltpu.async_copy(
                    utab_hbm.at[uidx_v.at[row]],
                    urows_v.at[pl.ds(c * L, L)], sem))
                copies.append(pltpu.async_copy(
                    mtab_hbm.at[midx_v.at[row]],
                    mrows_v.at[pl.ds(c * L, L)], sem))
            for cp in copies:
                cp.wait()
            pltpu.sync_copy(urows_v, uout_hbm.at[pl.ds(base + h * half, half)])
            pltpu.sync_copy(mrows_v, mout_hbm.at[pl.ds(base + h * half, half)])

    return gather_kernel(uid2d, mid2d, utab_pad, mtab_pad)


def _mlp_body(u_ref, m_ref, w1u_ref, w1m_ref, b1_ref, w2_ref, b2_ref,
              w3_ref, b3_ref, o_ref):
    h = jnp.dot(u_ref[...], w1u_ref[...], preferred_element_type=jnp.float32)
    h = h + jnp.dot(m_ref[...], w1m_ref[...], preferred_element_type=jnp.float32)
    h = jnp.maximum(h + b1_ref[...], 0.0)
    h = jnp.maximum(
        jnp.dot(h, w2_ref[...], preferred_element_type=jnp.float32)
        + b2_ref[...], 0.0)
    o_ref[...] = (jnp.dot(h, w3_ref[...], preferred_element_type=jnp.float32)
                  + b3_ref[...])


def _tc_mlp(upad, mpad, W1u, W1m, b1r, W2, b2r, W3, b3r):
    B, DP = upad.shape
    H = W2.shape[0]
    H2 = W2.shape[1]
    BB = 2048
    grid = (B // BB,)
    full = lambda shape: pl.BlockSpec(shape, lambda i: (0, 0))
    return pl.pallas_call(
        _mlp_body,
        grid=grid,
        in_specs=[
            pl.BlockSpec((BB, DP), lambda i: (i, 0)),
            pl.BlockSpec((BB, DP), lambda i: (i, 0)),
            full((DP, H)),
            full((DP, H)),
            full((1, H)),
            full((H, H2)),
            full((1, H2)),
            full((H2, 1)),
            full((1, 1)),
        ],
        out_specs=pl.BlockSpec((BB, 1), lambda i: (i, 0)),
        out_shape=jax.ShapeDtypeStruct((B, 1), jnp.float32),
    )(upad, mpad, W1u, W1m, b1r, W2, b2r, W3, b3r)


def kernel(user_ids, movie_ids, user_table, movie_table, W1, b1, W2, b2, W3, b3):
    D = user_table.shape[1]
    pad = _LANES - D
    uid2d = user_ids.astype(jnp.int32).reshape(-1, _LANES)
    mid2d = movie_ids.astype(jnp.int32).reshape(-1, _LANES)
    utab_pad = jnp.pad(user_table, ((0, 0), (0, pad)))
    mtab_pad = jnp.pad(movie_table, ((0, 0), (0, pad)))
    upad, mpad = _sc_gather(uid2d, mid2d, utab_pad, mtab_pad)
    W1u = jnp.pad(W1[:D], ((0, pad), (0, 0)))
    W1m = jnp.pad(W1[D:], ((0, pad), (0, 0)))
    out = _tc_mlp(upad, mpad, W1u, W1m, b1.reshape(1, -1),
                  W2, b2.reshape(1, -1), W3, b3.reshape(1, 1))
    return out


# speedup vs baseline: 1.6501x; 1.4764x over previous
import functools

import jax
import jax.numpy as jnp
from jax import lax
from jax.experimental import pallas as pl
from jax.experimental.pallas import tpu as pltpu
from jax.experimental.pallas import tpu_sc as plsc

_LANES = 128


def _xpose_body(inT_ref, out_ref):
    x = inT_ref[...]
    xt = x.T
    d = xt.shape[1]
    out_ref[...] = jnp.concatenate(
        [xt, jnp.zeros((xt.shape[0], _LANES - d), jnp.float32)], axis=1)


def _transpose_pad(tabT):
    D, N = tabT.shape
    BB = 4096
    grid = (pl.cdiv(N, BB),)
    return pl.pallas_call(
        _xpose_body,
        grid=grid,
        in_specs=[pl.BlockSpec((D, BB), lambda i: (0, i))],
        out_specs=pl.BlockSpec((BB, _LANES), lambda i: (i, 0)),
        out_shape=jax.ShapeDtypeStruct((N, _LANES), jnp.float32),
    )(tabT)


def _sc_gather(uid2d, mid2d, utab_pad, mtab_pad):
    info = plsc.get_sparse_core_info()
    NC, NS = info.num_cores, info.num_subcores
    NW = NC * NS
    n_rows, L = uid2d.shape
    B = n_rows * L
    rows_per_w = n_rows // NW
    b_per_w = B // NW
    half = b_per_w // 2
    mesh = plsc.VectorSubcoreMesh(core_axis_name="c", subcore_axis_name="s")

    @functools.partial(
        pl.kernel,
        mesh=mesh,
        out_type=[
            jax.ShapeDtypeStruct((B, _LANES), jnp.float32),
            jax.ShapeDtypeStruct((B, _LANES), jnp.float32),
        ],
        scratch_types=[
            pltpu.VMEM((rows_per_w, L), jnp.int32),
            pltpu.VMEM((rows_per_w, L), jnp.int32),
            pltpu.VMEM((half, _LANES), jnp.float32),
            pltpu.VMEM((half, _LANES), jnp.float32),
            pltpu.SemaphoreType.DMA,
        ],
        compiler_params=pltpu.CompilerParams(use_tc_tiling_on_sc=True),
    )
    def gather_kernel(uid_hbm, mid_hbm, utab_hbm, mtab_hbm,
                      uout_hbm, mout_hbm,
                      uidx_v, midx_v, urows_v, mrows_v, sem):
        wid = lax.axis_index("s") * NC + lax.axis_index("c")
        rbase = wid * rows_per_w
        base = wid * b_per_w
        pltpu.sync_copy(uid_hbm.at[pl.ds(rbase, rows_per_w)], uidx_v)
        pltpu.sync_copy(mid_hbm.at[pl.ds(rbase, rows_per_w)], midx_v)
        chunks_per_half = half // L
        for h in range(2):
            copies = []
            for c in range(chunks_per_half):
                row = h * chunks_per_half + c
                copies.append(pltpu.async_copy(
                    utab_hbm.at[uidx_v.at[row]],
                    urows_v.at[pl.ds(c * L, L)], sem))
                copies.append(pltpu.async_copy(
                    mtab_hbm.at[midx_v.at[row]],
                    mrows_v.at[pl.ds(c * L, L)], sem))
            for cp in copies:
                cp.wait()
            pltpu.sync_copy(urows_v, uout_hbm.at[pl.ds(base + h * half, half)])
            pltpu.sync_copy(mrows_v, mout_hbm.at[pl.ds(base + h * half, half)])

    return gather_kernel(uid2d, mid2d, utab_pad, mtab_pad)


def _mlp_body(u_ref, m_ref, w1u_ref, w1m_ref, b1_ref, w2_ref, b2_ref,
              w3_ref, b3_ref, o_ref):
    h = jnp.dot(u_ref[...], w1u_ref[...], preferred_element_type=jnp.float32)
    h = h + jnp.dot(m_ref[...], w1m_ref[...], preferred_element_type=jnp.float32)
    h = jnp.maximum(h + b1_ref[...], 0.0)
    h = jnp.maximum(
        jnp.dot(h, w2_ref[...], preferred_element_type=jnp.float32)
        + b2_ref[...], 0.0)
    o_ref[...] = (jnp.dot(h, w3_ref[...], preferred_element_type=jnp.float32)
                  + b3_ref[...])


def _tc_mlp(upad, mpad, W1u, W1m, b1r, W2, b2r, W3, b3r):
    B, DP = upad.shape
    H = W2.shape[0]
    H2 = W2.shape[1]
    BB = 2048
    grid = (B // BB,)
    full = lambda shape: pl.BlockSpec(shape, lambda i: (0, 0))
    return pl.pallas_call(
        _mlp_body,
        grid=grid,
        in_specs=[
            pl.BlockSpec((BB, DP), lambda i: (i, 0)),
            pl.BlockSpec((BB, DP), lambda i: (i, 0)),
            full((DP, H)),
            full((DP, H)),
            full((1, H)),
            full((H, H2)),
            full((1, H2)),
            full((H2, 1)),
            full((1, 1)),
        ],
        out_specs=pl.BlockSpec((BB, 1), lambda i: (i, 0)),
        out_shape=jax.ShapeDtypeStruct((B, 1), jnp.float32),
    )(upad, mpad, W1u, W1m, b1r, W2, b2r, W3, b3r)


def kernel(user_ids, movie_ids, user_table, movie_table, W1, b1, W2, b2, W3, b3):
    D = user_table.shape[1]
    pad = _LANES - D
    uid2d = user_ids.astype(jnp.int32).reshape(-1, _LANES)
    mid2d = movie_ids.astype(jnp.int32).reshape(-1, _LANES)
    utab_pad = _transpose_pad(user_table.T)
    mtab_pad = _transpose_pad(movie_table.T)
    upad, mpad = _sc_gather(uid2d, mid2d, utab_pad, mtab_pad)
    W1u = jnp.pad(W1[:D], ((0, pad), (0, 0)))
    W1m = jnp.pad(W1[D:], ((0, pad), (0, 0)))
    out = _tc_mlp(upad, mpad, W1u, W1m, b1.reshape(1, -1),
                  W2, b2.reshape(1, -1), W3, b3.reshape(1, 1))
    return out


# speedup vs baseline: 2.0439x; 1.2387x over previous
import functools

import jax
import jax.numpy as jnp
from jax import lax
from jax.experimental import pallas as pl
from jax.experimental.pallas import tpu as pltpu
from jax.experimental.pallas import tpu_sc as plsc

_LANES = 128


def _xpose_body(inT_ref, out_ref):
    x = inT_ref[...]
    xt = x.T
    d = xt.shape[1]
    out_ref[...] = jnp.concatenate(
        [xt, jnp.zeros((xt.shape[0], _LANES - d), jnp.float32)], axis=1)


def _transpose_pad(tabT):
    D, N = tabT.shape
    BB = 8192
    grid = (pl.cdiv(N, BB),)
    return pl.pallas_call(
        _xpose_body,
        grid=grid,
        in_specs=[pl.BlockSpec((D, BB), lambda i: (0, i))],
        out_specs=pl.BlockSpec((BB, _LANES), lambda i: (i, 0)),
        out_shape=jax.ShapeDtypeStruct((N, _LANES), jnp.float32),
    )(tabT)


def _sc_gather(uid2d, mid2d, utab_pad, mtab_pad):
    info = plsc.get_sparse_core_info()
    NC, NS = info.num_cores, info.num_subcores
    NW = NC * NS
    n_rows, L = uid2d.shape
    B = n_rows * L
    rows_per_w = n_rows // NW
    b_per_w = B // NW
    half = b_per_w // 2
    mesh = plsc.VectorSubcoreMesh(core_axis_name="c", subcore_axis_name="s")

    @functools.partial(
        pl.kernel,
        mesh=mesh,
        out_type=[
            jax.ShapeDtypeStruct((B, _LANES), jnp.float32),
            jax.ShapeDtypeStruct((B, _LANES), jnp.float32),
        ],
        scratch_types=[
            pltpu.VMEM((rows_per_w, L), jnp.int32),
            pltpu.VMEM((rows_per_w, L), jnp.int32),
            pltpu.VMEM((half, _LANES), jnp.float32),
            pltpu.VMEM((half, _LANES), jnp.float32),
            pltpu.SemaphoreType.DMA,
        ],
        compiler_params=pltpu.CompilerParams(use_tc_tiling_on_sc=True),
    )
    def gather_kernel(uid_hbm, mid_hbm, utab_hbm, mtab_hbm,
                      uout_hbm, mout_hbm,
                      uidx_v, midx_v, urows_v, mrows_v, sem):
        wid = lax.axis_index("s") * NC + lax.axis_index("c")
        rbase = wid * rows_per_w
        base = wid * b_per_w
        pltpu.sync_copy(uid_hbm.at[pl.ds(rbase, rows_per_w)], uidx_v)
        pltpu.sync_copy(mid_hbm.at[pl.ds(rbase, rows_per_w)], midx_v)
        chunks_per_half = half // L
        for h in range(2):
            copies = []
            for c in range(chunks_per_half):
                row = h * chunks_per_half + c
                copies.append(pltpu.async_copy(
                    utab_hbm.at[uidx_v.at[row]],
                    urows_v.at[pl.ds(c * L, L)], sem))
                copies.append(pltpu.async_copy(
                    mtab_hbm.at[midx_v.at[row]],
                    mrows_v.at[pl.ds(c * L, L)], sem))
            for cp in copies:
                cp.wait()
            pltpu.sync_copy(urows_v, uout_hbm.at[pl.ds(base + h * half, half)])
            pltpu.sync_copy(mrows_v, mout_hbm.at[pl.ds(base + h * half, half)])

    return gather_kernel(uid2d, mid2d, utab_pad, mtab_pad)


def _mlp_body(u_ref, m_ref, w1u_ref, w1m_ref, b1_ref, w2_ref, b2_ref,
              w3_ref, b3_ref, o_ref):
    h = jnp.dot(u_ref[...], w1u_ref[...], preferred_element_type=jnp.float32)
    h = h + jnp.dot(m_ref[...], w1m_ref[...], preferred_element_type=jnp.float32)
    h = jnp.maximum(h + b1_ref[...], 0.0)
    h = jnp.maximum(
        jnp.dot(h, w2_ref[...], preferred_element_type=jnp.float32)
        + b2_ref[...], 0.0)
    o_ref[...] = (jnp.dot(h, w3_ref[...], preferred_element_type=jnp.float32)
                  + b3_ref[...])


def _tc_mlp(upad, mpad, W1u, W1m, b1r, W2, b2r, W3, b3r):
    B, DP = upad.shape
    H = W2.shape[0]
    H2 = W2.shape[1]
    BB = 4096
    grid = (B // BB,)
    full = lambda shape: pl.BlockSpec(shape, lambda i: (0, 0))
    return pl.pallas_call(
        _mlp_body,
        grid=grid,
        in_specs=[
            pl.BlockSpec((BB, DP), lambda i: (i, 0)),
            pl.BlockSpec((BB, DP), lambda i: (i, 0)),
            full((DP, H)),
            full((DP, H)),
            full((1, H)),
            full((H, H2)),
            full((1, H2)),
            full((H2, 1)),
            full((1, 1)),
        ],
        out_specs=pl.BlockSpec((BB, 1), lambda i: (i, 0)),
        out_shape=jax.ShapeDtypeStruct((B, 1), jnp.float32),
    )(upad, mpad, W1u, W1m, b1r, W2, b2r, W3, b3r)


def kernel(user_ids, movie_ids, user_table, movie_table, W1, b1, W2, b2, W3, b3):
    D = user_table.shape[1]
    pad = _LANES - D
    uid2d = user_ids.astype(jnp.int32).reshape(-1, _LANES)
    mid2d = movie_ids.astype(jnp.int32).reshape(-1, _LANES)
    utab_pad = _transpose_pad(user_table.T)
    mtab_pad = _transpose_pad(movie_table.T)
    upad, mpad = _sc_gather(uid2d, mid2d, utab_pad, mtab_pad)
    W1u = jnp.pad(W1[:D], ((0, pad), (0, 0)))
    W1m = jnp.pad(W1[D:], ((0, pad), (0, 0)))
    out = _tc_mlp(upad, mpad, W1u, W1m, b1.reshape(1, -1),
                  W2, b2.reshape(1, -1), W3, b3.reshape(1, 1))
    return out


# speedup vs baseline: 2.1847x; 1.0689x over previous
import functools

import jax
import jax.numpy as jnp
from jax import lax
from jax.experimental import pallas as pl
from jax.experimental.pallas import tpu as pltpu
from jax.experimental.pallas import tpu_sc as plsc

_LANES = 128


def _xpose_body(inT_ref, out_ref):
    x = inT_ref[...]
    xt = x.T
    d = xt.shape[1]
    out_ref[...] = jnp.concatenate(
        [xt, jnp.zeros((xt.shape[0], _LANES - d), jnp.float32)], axis=1)


def _transpose_pad(tabT):
    D, N = tabT.shape
    BB = 16384
    grid = (pl.cdiv(N, BB),)
    return pl.pallas_call(
        _xpose_body,
        grid=grid,
        in_specs=[pl.BlockSpec((D, BB), lambda i: (0, i))],
        out_specs=pl.BlockSpec((BB, _LANES), lambda i: (i, 0)),
        out_shape=jax.ShapeDtypeStruct((N, _LANES), jnp.float32),
    )(tabT)


def _sc_gather(uid2d, mid2d, utab_pad, mtab_pad):
    info = plsc.get_sparse_core_info()
    NC, NS = info.num_cores, info.num_subcores
    NW = NC * NS
    n_rows, L = uid2d.shape
    B = n_rows * L
    rows_per_w = n_rows // NW
    b_per_w = B // NW
    half = b_per_w // 2
    mesh = plsc.VectorSubcoreMesh(core_axis_name="c", subcore_axis_name="s")

    @functools.partial(
        pl.kernel,
        mesh=mesh,
        out_type=[
            jax.ShapeDtypeStruct((B, _LANES), jnp.float32),
            jax.ShapeDtypeStruct((B, _LANES), jnp.float32),
        ],
        scratch_types=[
            pltpu.VMEM((rows_per_w, L), jnp.int32),
            pltpu.VMEM((rows_per_w, L), jnp.int32),
            pltpu.VMEM((half, _LANES), jnp.float32),
            pltpu.VMEM((half, _LANES), jnp.float32),
            pltpu.SemaphoreType.DMA,
        ],
        compiler_params=pltpu.CompilerParams(use_tc_tiling_on_sc=True),
    )
    def gather_kernel(uid_hbm, mid_hbm, utab_hbm, mtab_hbm,
                      uout_hbm, mout_hbm,
                      uidx_v, midx_v, urows_v, mrows_v, sem):
        wid = lax.axis_index("s") * NC + lax.axis_index("c")
        rbase = wid * rows_per_w
        base = wid * b_per_w
        pltpu.sync_copy(uid_hbm.at[pl.ds(rbase, rows_per_w)], uidx_v)
        pltpu.sync_copy(mid_hbm.at[pl.ds(rbase, rows_per_w)], midx_v)
        chunks_per_half = half // L
        for h in range(2):
            copies = []
            for c in range(chunks_per_half):
                row = h * chunks_per_half + c
                copies.append(pltpu.async_copy(
                    utab_hbm.at[uidx_v.at[row]],
                    urows_v.at[pl.ds(c * L, L)], sem))
                copies.append(pltpu.async_copy(
                    mtab_hbm.at[midx_v.at[row]],
                    mrows_v.at[pl.ds(c * L, L)], sem))
            for cp in copies:
                cp.wait()
            pltpu.sync_copy(urows_v, uout_hbm.at[pl.ds(base + h * half, half)])
            pltpu.sync_copy(mrows_v, mout_hbm.at[pl.ds(base + h * half, half)])

    return gather_kernel(uid2d, mid2d, utab_pad, mtab_pad)


def _mlp_body(u_ref, m_ref, w1u_ref, w1m_ref, b1_ref, w2_ref, b2_ref,
              w3_ref, b3_ref, o_ref):
    h = jnp.dot(u_ref[...], w1u_ref[...], preferred_element_type=jnp.float32)
    h = h + jnp.dot(m_ref[...], w1m_ref[...], preferred_element_type=jnp.float32)
    h = jnp.maximum(h + b1_ref[...], 0.0)
    h = jnp.maximum(
        jnp.dot(h, w2_ref[...], preferred_element_type=jnp.float32)
        + b2_ref[...], 0.0)
    o_ref[...] = (jnp.dot(h, w3_ref[...], preferred_element_type=jnp.float32)
                  + b3_ref[...])


def _tc_mlp(upad, mpad, W1u, W1m, b1r, W2, b2r, W3, b3r):
    B, DP = upad.shape
    H = W2.shape[0]
    H2 = W2.shape[1]
    BB = 4096
    grid = (B // BB,)
    full = lambda shape: pl.BlockSpec(shape, lambda i: (0, 0))
    return pl.pallas_call(
        _mlp_body,
        grid=grid,
        in_specs=[
            pl.BlockSpec((BB, DP), lambda i: (i, 0)),
            pl.BlockSpec((BB, DP), lambda i: (i, 0)),
            full((DP, H)),
            full((DP, H)),
            full((1, H)),
            full((H, H2)),
            full((1, H2)),
            full((H2, 1)),
            full((1, 1)),
        ],
        out_specs=pl.BlockSpec((BB, 1), lambda i: (i, 0)),
        out_shape=jax.ShapeDtypeStruct((B, 1), jnp.float32),
    )(upad, mpad, W1u, W1m, b1r, W2, b2r, W3, b3r)


def kernel(user_ids, movie_ids, user_table, movie_table, W1, b1, W2, b2, W3, b3):
    D = user_table.shape[1]
    pad = _LANES - D
    uid2d = user_ids.astype(jnp.int32).reshape(-1, _LANES)
    mid2d = movie_ids.astype(jnp.int32).reshape(-1, _LANES)
    utab_pad = _transpose_pad(user_table.T)
    mtab_pad = _transpose_pad(movie_table.T)
    upad, mpad = _sc_gather(uid2d, mid2d, utab_pad, mtab_pad)
    W1u = jnp.pad(W1[:D], ((0, pad), (0, 0)))
    W1m = jnp.pad(W1[D:], ((0, pad), (0, 0)))
    out = _tc_mlp(upad, mpad, W1u, W1m, b1.reshape(1, -1),
                  W2, b2.reshape(1, -1), W3, b3.reshape(1, 1))
    return out


# speedup vs baseline: 2.2000x; 1.0070x over previous
import functools

import jax
import jax.numpy as jnp
from jax import lax
from jax.experimental import pallas as pl
from jax.experimental.pallas import tpu as pltpu
from jax.experimental.pallas import tpu_sc as plsc

_LANES = 128
_XBB = 16384


def _xpose_body(inT_ref, out_ref):
    x = inT_ref[...]
    xt = x.T
    h = xt.shape[0] // 2
    out_ref[...] = jnp.concatenate([xt[:h], xt[h:]], axis=1)


def _transpose_pack(tabT):
    D, N = tabT.shape
    nblk = pl.cdiv(N, _XBB)
    half = _XBB // 2
    return pl.pallas_call(
        _xpose_body,
        grid=(nblk,),
        in_specs=[pl.BlockSpec((D, _XBB), lambda i: (0, i))],
        out_specs=pl.BlockSpec((half, 2 * D), lambda i: (i, 0)),
        out_shape=jax.ShapeDtypeStruct((nblk * half, 2 * D), jnp.float32),
    )(tabT)


def _pack_row_ids(ids):
    blk = ids // _XBB
    t = ids % _XBB
    half = _XBB // 2
    row = blk * half + t % half
    sel = t // half
    return row, sel


def _sc_gather(uid2d, mid2d, utab_pack, mtab_pack):
    info = plsc.get_sparse_core_info()
    NC, NS = info.num_cores, info.num_subcores
    NW = NC * NS
    n_rows, L = uid2d.shape
    B = n_rows * L
    rows_per_w = n_rows // NW
    b_per_w = B // NW
    half = b_per_w // 2
    mesh = plsc.VectorSubcoreMesh(core_axis_name="c", subcore_axis_name="s")

    @functools.partial(
        pl.kernel,
        mesh=mesh,
        out_type=[
            jax.ShapeDtypeStruct((B, _LANES), jnp.float32),
            jax.ShapeDtypeStruct((B, _LANES), jnp.float32),
        ],
        scratch_types=[
            pltpu.VMEM((rows_per_w, L), jnp.int32),
            pltpu.VMEM((rows_per_w, L), jnp.int32),
            pltpu.VMEM((half, _LANES), jnp.float32),
            pltpu.VMEM((half, _LANES), jnp.float32),
            pltpu.SemaphoreType.DMA,
        ],
        compiler_params=pltpu.CompilerParams(use_tc_tiling_on_sc=True),
    )
    def gather_kernel(uid_hbm, mid_hbm, utab_hbm, mtab_hbm,
                      uout_hbm, mout_hbm,
                      uidx_v, midx_v, urows_v, mrows_v, sem):
        wid = lax.axis_index("s") * NC + lax.axis_index("c")
        rbase = wid * rows_per_w
        base = wid * b_per_w
        pltpu.sync_copy(uid_hbm.at[pl.ds(rbase, rows_per_w)], uidx_v)
        pltpu.sync_copy(mid_hbm.at[pl.ds(rbase, rows_per_w)], midx_v)
        chunks_per_half = half // L
        for h in range(2):
            copies = []
            for c in range(chunks_per_half):
                row = h * chunks_per_half + c
                copies.append(pltpu.async_copy(
                    utab_hbm.at[uidx_v.at[row]],
                    urows_v.at[pl.ds(c * L, L)], sem))
                copies.append(pltpu.async_copy(
                    mtab_hbm.at[midx_v.at[row]],
                    mrows_v.at[pl.ds(c * L, L)], sem))
            for cp in copies:
                cp.wait()
            pltpu.sync_copy(urows_v, uout_hbm.at[pl.ds(base + h * half, half)])
            pltpu.sync_copy(mrows_v, mout_hbm.at[pl.ds(base + h * half, half)])

    return gather_kernel(uid2d, mid2d, utab_pack, mtab_pack)


def _mlp_body(u_ref, m_ref, usel_ref, msel_ref, w1u_ref, w1m_ref, b1_ref,
              w2_ref, b2_ref, w3_ref, b3_ref, o_ref):
    d = w1u_ref.shape[0]
    urows = u_ref[...]
    mrows = m_ref[...]
    u = jnp.where(usel_ref[...] > 0, urows[:, d:], urows[:, :d])
    m = jnp.where(msel_ref[...] > 0, mrows[:, d:], mrows[:, :d])
    h = jnp.dot(u, w1u_ref[...], preferred_element_type=jnp.float32)
    h = h + jnp.dot(m, w1m_ref[...], preferred_element_type=jnp.float32)
    h = jnp.maximum(h + b1_ref[...], 0.0)
    h = jnp.maximum(
        jnp.dot(h, w2_ref[...], preferred_element_type=jnp.float32)
        + b2_ref[...], 0.0)
    o_ref[...] = (jnp.dot(h, w3_ref[...], preferred_element_type=jnp.float32)
                  + b3_ref[...])


def _tc_mlp(upack, mpack, usel, msel, W1u, W1m, b1r, W2, b2r, W3, b3r):
    B = upack.shape[0]
    D = W1u.shape[0]
    H = W2.shape[0]
    H2 = W2.shape[1]
    BB = 4096
    grid = (B // BB,)
    full = lambda shape: pl.BlockSpec(shape, lambda i: (0, 0))
    return pl.pallas_call(
        _mlp_body,
        grid=grid,
        in_specs=[
            pl.BlockSpec((BB, 2 * D), lambda i: (i, 0)),
            pl.BlockSpec((BB, 2 * D), lambda i: (i, 0)),
            pl.BlockSpec((BB, 1), lambda i: (i, 0)),
            pl.BlockSpec((BB, 1), lambda i: (i, 0)),
            full((D, H)),
            full((D, H)),
            full((1, H)),
            full((H, H2)),
            full((1, H2)),
            full((H2, 1)),
            full((1, 1)),
        ],
        out_specs=pl.BlockSpec((BB, 1), lambda i: (i, 0)),
        out_shape=jax.ShapeDtypeStruct((B, 1), jnp.float32),
    )(upack, mpack, usel, msel, W1u, W1m, b1r, W2, b2r, W3, b3r)


def kernel(user_ids, movie_ids, user_table, movie_table, W1, b1, W2, b2, W3, b3):
    D = user_table.shape[1]
    B = user_ids.shape[0]
    uids = user_ids.astype(jnp.int32)
    mids = movie_ids.astype(jnp.int32)
    urow, usel = _pack_row_ids(uids)
    mrow, msel = _pack_row_ids(mids)
    uid2d = urow.reshape(-1, _LANES)
    mid2d = mrow.reshape(-1, _LANES)
    utab_pack = _transpose_pack(user_table.T)
    mtab_pack = _transpose_pack(movie_table.T)
    upack, mpack = _sc_gather(uid2d, mid2d, utab_pack, mtab_pack)
    out = _tc_mlp(upack, mpack, usel.reshape(B, 1), msel.reshape(B, 1),
                  W1[:D], W1[D:], b1.reshape(1, -1),
                  W2, b2.reshape(1, -1), W3, b3.reshape(1, 1))
    return out


# speedup vs baseline: 2.2075x; 1.0034x over previous
import functools

import jax
import jax.numpy as jnp
from jax import lax
from jax.experimental import pallas as pl
from jax.experimental.pallas import tpu as pltpu
from jax.experimental.pallas import tpu_sc as plsc

_LANES = 128
_XBB = 16384


def _xpose_body(inT_ref, eye_ref, out_ref):
    x = inT_ref[...]
    dn = (((0,), (0,)), ((), ()))
    xt = lax.dot_general(x, eye_ref[...], dn,
                         preferred_element_type=jnp.float32)
    h = xt.shape[0] // 2
    out_ref[...] = jnp.concatenate([xt[:h], xt[h:]], axis=1)


def _transpose_pack(tabT):
    D, N = tabT.shape
    nblk = pl.cdiv(N, _XBB)
    half = _XBB // 2
    return pl.pallas_call(
        _xpose_body,
        grid=(nblk,),
        in_specs=[pl.BlockSpec((D, _XBB), lambda i: (0, i)),
                  pl.BlockSpec((D, D), lambda i: (0, 0))],
        out_specs=pl.BlockSpec((half, 2 * D), lambda i: (i, 0)),
        out_shape=jax.ShapeDtypeStruct((nblk * half, 2 * D), jnp.float32),
    )(tabT, jnp.eye(D, dtype=jnp.float32))


def _pack_row_ids(ids):
    blk = ids // _XBB
    t = ids % _XBB
    half = _XBB // 2
    row = blk * half + t % half
    sel = t // half
    return row, sel


def _sc_gather(uid2d, mid2d, utab_pack, mtab_pack):
    info = plsc.get_sparse_core_info()
    NC, NS = info.num_cores, info.num_subcores
    NW = NC * NS
    n_rows, L = uid2d.shape
    B = n_rows * L
    rows_per_w = n_rows // NW
    b_per_w = B // NW
    half = b_per_w // 2
    mesh = plsc.VectorSubcoreMesh(core_axis_name="c", subcore_axis_name="s")

    @functools.partial(
        pl.kernel,
        mesh=mesh,
        out_type=[
            jax.ShapeDtypeStruct((B, _LANES), jnp.float32),
            jax.ShapeDtypeStruct((B, _LANES), jnp.float32),
        ],
        scratch_types=[
            pltpu.VMEM((rows_per_w, L), jnp.int32),
            pltpu.VMEM((rows_per_w, L), jnp.int32),
            pltpu.VMEM((half, _LANES), jnp.float32),
            pltpu.VMEM((half, _LANES), jnp.float32),
            pltpu.SemaphoreType.DMA,
        ],
        compiler_params=pltpu.CompilerParams(use_tc_tiling_on_sc=True),
    )
    def gather_kernel(uid_hbm, mid_hbm, utab_hbm, mtab_hbm,
                      uout_hbm, mout_hbm,
                      uidx_v, midx_v, urows_v, mrows_v, sem):
        wid = lax.axis_index("s") * NC + lax.axis_index("c")
        rbase = wid * rows_per_w
        base = wid * b_per_w
        pltpu.sync_copy(uid_hbm.at[pl.ds(rbase, rows_per_w)], uidx_v)
        pltpu.sync_copy(mid_hbm.at[pl.ds(rbase, rows_per_w)], midx_v)
        chunks_per_half = half // L
        for h in range(2):
            copies = []
            for c in range(chunks_per_half):
                row = h * chunks_per_half + c
                copies.append(pltpu.async_copy(
                    utab_hbm.at[uidx_v.at[row]],
                    urows_v.at[pl.ds(c * L, L)], sem))
                copies.append(pltpu.async_copy(
                    mtab_hbm.at[midx_v.at[row]],
                    mrows_v.at[pl.ds(c * L, L)], sem))
            for cp in copies:
                cp.wait()
            pltpu.sync_copy(urows_v, uout_hbm.at[pl.ds(base + h * half, half)])
            pltpu.sync_copy(mrows_v, mout_hbm.at[pl.ds(base + h * half, half)])

    return gather_kernel(uid2d, mid2d, utab_pack, mtab_pack)


def _mlp_body(u_ref, m_ref, usel_ref, msel_ref, w1u_ref, w1m_ref, b1_ref,
              w2_ref, b2_ref, w3_ref, b3_ref, o_ref):
    d = w1u_ref.shape[0]
    urows = u_ref[...]
    mrows = m_ref[...]
    u = jnp.where(usel_ref[...] > 0, urows[:, d:], urows[:, :d])
    m = jnp.where(msel_ref[...] > 0, mrows[:, d:], mrows[:, :d])
    h = jnp.dot(u, w1u_ref[...], preferred_element_type=jnp.float32)
    h = h + jnp.dot(m, w1m_ref[...], preferred_element_type=jnp.float32)
    h = jnp.maximum(h + b1_ref[...], 0.0)
    h = jnp.maximum(
        jnp.dot(h, w2_ref[...], preferred_element_type=jnp.float32)
        + b2_ref[...], 0.0)
    o_ref[...] = (jnp.dot(h, w3_ref[...], preferred_element_type=jnp.float32)
                  + b3_ref[...])


def _tc_mlp(upack, mpack, usel, msel, W1u, W1m, b1r, W2, b2r, W3, b3r):
    B = upack.shape[0]
    D = W1u.shape[0]
    H = W2.shape[0]
    H2 = W2.shape[1]
    BB = 4096
    grid = (B // BB,)
    full = lambda shape: pl.BlockSpec(shape, lambda i: (0, 0))
    return pl.pallas_call(
        _mlp_body,
        grid=grid,
        in_specs=[
            pl.BlockSpec((BB, 2 * D), lambda i: (i, 0)),
            pl.BlockSpec((BB, 2 * D), lambda i: (i, 0)),
            pl.BlockSpec((BB, 1), lambda i: (i, 0)),
            pl.BlockSpec((BB, 1), lambda i: (i, 0)),
            full((D, H)),
            full((D, H)),
            full((1, H)),
            full((H, H2)),
            full((1, H2)),
            full((H2, 1)),
            full((1, 1)),
        ],
        out_specs=pl.BlockSpec((BB, 1), lambda i: (i, 0)),
        out_shape=jax.ShapeDtypeStruct((B, 1), jnp.float32),
    )(upack, mpack, usel, msel, W1u, W1m, b1r, W2, b2r, W3, b3r)


def kernel(user_ids, movie_ids, user_table, movie_table, W1, b1, W2, b2, W3, b3):
    D = user_table.shape[1]
    B = user_ids.shape[0]
    uids = user_ids.astype(jnp.int32)
    mids = movie_ids.astype(jnp.int32)
    urow, usel = _pack_row_ids(uids)
    mrow, msel = _pack_row_ids(mids)
    uid2d = urow.reshape(-1, _LANES)
    mid2d = mrow.reshape(-1, _LANES)
    utab_pack = _transpose_pack(user_table.T)
    mtab_pack = _transpose_pack(movie_table.T)
    upack, mpack = _sc_gather(uid2d, mid2d, utab_pack, mtab_pack)
    out = _tc_mlp(upack, mpack, usel.reshape(B, 1), msel.reshape(B, 1),
                  W1[:D], W1[D:], b1.reshape(1, -1),
                  W2, b2.reshape(1, -1), W3, b3.reshape(1, 1))
    return out
